# Initial kernel scaffold; baseline (speedup 1.0000x reference)
import functools

import jax
import jax.numpy as jnp
from jax import lax
from jax.experimental import pallas as pl
from jax.experimental.pallas import tpu as pltpu
from jax.experimental.pallas import tpu_sc as plsc

NC = 2
NS = 16
NW = NC * NS

G = 100
K = 8
C = G * K


def _tok_pos_kernel(BF, L, E, per_w, n_chunks):
    mesh = plsc.VectorSubcoreMesh(core_axis_name="c", subcore_axis_name="s")

    @functools.partial(
        pl.kernel,
        mesh=mesh,
        out_type=jax.ShapeDtypeStruct((BF, E), jnp.float32),
        scratch_types=[
            pltpu.VMEM((K, G), jnp.int32),
            pltpu.VMEM((C, E), jnp.float32),
            pltpu.VMEM((L, E), jnp.float32),
            pltpu.SemaphoreType.DMA,
        ],
    )
    def k(x_hbm, tok_hbm, pos_hbm, out_hbm, idx_v, rows_v, pos_v, gsem):
        wid = lax.axis_index("s") * NC + lax.axis_index("c")
        base = wid * per_w
        pltpu.sync_copy(pos_hbm, pos_v)

        def chunk_body(c, carry):
            row0 = base + c * C
            pltpu.sync_copy(x_hbm.at[pl.ds(row0 // G, K)], idx_v)
            copies = []
            for j in range(K):
                copies.append(
                    pltpu.async_copy(
                        tok_hbm.at[idx_v.at[j]],
                        rows_v.at[pl.ds(j * G, G)],
                        gsem,
                    )
                )
            for cp in copies:
                cp.wait()

            def add_body(p, acc):
                for e in range(E // 16):
                    pv = pos_v[p, pl.ds(e * 16, 16)]
                    for s in range(C // L):
                        r = s * L + p
                        rows_v[r, pl.ds(e * 16, 16)] = (
                            rows_v[r, pl.ds(e * 16, 16)] + pv
                        )
                return acc

            lax.fori_loop(0, L, add_body, 0)
            pltpu.sync_copy(rows_v, out_hbm.at[pl.ds(row0, C)])
            return carry

        lax.fori_loop(0, n_chunks, chunk_body, 0)

    return k


def kernel(x, token_table, pos_table):
    B, L = x.shape
    V, E = token_table.shape
    BF = B * L
    per_w = BF // NW
    n_chunks = per_w // C

    xf = x.reshape(BF).astype(jnp.int32)
    x2 = xf.reshape(BF // G, G)

    k = _tok_pos_kernel(BF, L, E, per_w, n_chunks)
    out = k(x2, token_table, pos_table)
    return out.reshape(B, L, E)


# speedup vs baseline: 1.3893x; 1.3893x over previous
import functools

import jax
import jax.numpy as jnp
from jax import lax
from jax.experimental import pallas as pl
from jax.experimental.pallas import tpu as pltpu
from jax.experimental.pallas import tpu_sc as plsc

NC = 2
NS = 16
NW = NC * NS

G = 100
K = 8
C = G * K


def _tok_pos_kernel(BF, L, E, per_w, n_chunks):
    mesh = plsc.VectorSubcoreMesh(core_axis_name="c", subcore_axis_name="s")

    @functools.partial(
        pl.kernel,
        mesh=mesh,
        out_type=jax.ShapeDtypeStruct((BF, E), jnp.float32),
        compiler_params=pltpu.CompilerParams(use_tc_tiling_on_sc=False),
        scratch_types=[
            pltpu.VMEM((K, G), jnp.int32),
            pltpu.VMEM((C, E), jnp.float32),
            pltpu.VMEM((L, E), jnp.float32),
            pltpu.SemaphoreType.DMA,
        ],
    )
    def k(x_hbm, tok_hbm, pos_hbm, out_hbm, idx_v, rows_v, pos_v, gsem):
        wid = lax.axis_index("s") * NC + lax.axis_index("c")
        base = wid * per_w
        pltpu.sync_copy(pos_hbm, pos_v)

        def chunk_body(c, carry):
            row0 = pl.multiple_of(base + c * C, C)
            pltpu.sync_copy(x_hbm.at[pl.ds(pl.multiple_of(row0 // G, K), K)], idx_v)
            copies = []
            for j in range(K):
                copies.append(
                    pltpu.async_copy(
                        tok_hbm.at[idx_v.at[j]],
                        rows_v.at[pl.ds(j * G, G)],
                        gsem,
                    )
                )
            for cp in copies:
                cp.wait()

            def add_body(p, acc):
                for e in range(E // 16):
                    pv = pos_v[p, pl.ds(e * 16, 16)]
                    for s in range(C // L):
                        r = s * L + p
                        rows_v[r, pl.ds(e * 16, 16)] = (
                            rows_v[r, pl.ds(e * 16, 16)] + pv
                        )
                return acc

            lax.fori_loop(0, L, add_body, 0)
            pltpu.sync_copy(rows_v, out_hbm.at[pl.ds(row0, C)])
            return carry

        lax.fori_loop(0, n_chunks, chunk_body, 0)

    return k


def kernel(x, token_table, pos_table):
    B, L = x.shape
    V, E = token_table.shape
    BF = B * L
    per_w = BF // NW
    n_chunks = per_w // C

    xf = x.reshape(BF).astype(jnp.int32)
    x2 = xf.reshape(BF // G, G)

    k = _tok_pos_kernel(BF, L, E, per_w, n_chunks)
    out = k(x2, token_table, pos_table)
    return out.reshape(B, L, E)


# speedup vs baseline: 1.4825x; 1.0671x over previous
import functools

import jax
import jax.numpy as jnp
from jax import lax
from jax.experimental import pallas as pl
from jax.experimental.pallas import tpu as pltpu
from jax.experimental.pallas import tpu_sc as plsc

NC = 2
NS = 16
NW = NC * NS

G = 100
K = 4
C = G * K
NBUF = 4


def _tok_pos_kernel(BF, L, E, per_w, n_chunks):
    mesh = plsc.VectorSubcoreMesh(core_axis_name="c", subcore_axis_name="s")
    idx_rows = per_w // G

    scratch = (
        [pltpu.VMEM((idx_rows, G), jnp.int32)]
        + [pltpu.VMEM((C, E), jnp.float32) for _ in range(NBUF)]
        + [pltpu.VMEM((L, E), jnp.float32)]
        + [pltpu.SemaphoreType.DMA for _ in range(2 * NBUF)]
    )

    @functools.partial(
        pl.kernel,
        mesh=mesh,
        out_type=jax.ShapeDtypeStruct((BF, E), jnp.float32),
        compiler_params=pltpu.CompilerParams(use_tc_tiling_on_sc=False),
        scratch_types=scratch,
    )
    def k(x_hbm, tok_hbm, pos_hbm, out_hbm, idx_v, *rest):
        rows = rest[:NBUF]
        pos_v = rest[NBUF]
        gsem = rest[NBUF + 1:NBUF + 1 + NBUF]
        ssem = rest[NBUF + 1 + NBUF:]

        wid = lax.axis_index("s") * NC + lax.axis_index("c")
        base = wid * per_w
        pltpu.sync_copy(pos_hbm, pos_v)
        pltpu.sync_copy(
            x_hbm.at[pl.ds(pl.multiple_of(wid * idx_rows, 8), idx_rows)], idx_v
        )

        def gather_descr(c, b):
            return [
                pltpu.make_async_copy(
                    tok_hbm.at[idx_v.at[c * K + j]],
                    rows[b].at[pl.ds(j * G, G)],
                    gsem[b],
                )
                for j in range(K)
            ]

        def store_descr(c, b):
            row0 = pl.multiple_of(base + c * C, C)
            return pltpu.make_async_copy(rows[b], out_hbm.at[pl.ds(row0, C)], ssem[b])

        for b in range(NBUF - 1):
            for d in gather_descr(b, b):
                d.start()

        def outer(t, carry):
            for phase in range(NBUF):
                c = t * NBUF + phase
                b = phase
                bn = (phase + NBUF - 1) % NBUF

                @pl.when(c + NBUF - 1 < n_chunks)
                def _fire():
                    @pl.when(c >= 1)
                    def _drain():
                        store_descr(c - 1, bn).wait()

                    for d in gather_descr(c + NBUF - 1, bn):
                        d.start()

                for d in gather_descr(c, b):
                    d.wait()

                rv = rows[b]

                def add_body(i, acc):
                    for u in range(2):
                        p = i * 2 + u
                        pv0 = pos_v[p, pl.ds(0, 16)]
                        pv1 = pos_v[p, pl.ds(16, 16)]
                        for s in range(C // L):
                            r = s * L + p
                            rv[r, pl.ds(0, 16)] = rv[r, pl.ds(0, 16)] + pv0
                            rv[r, pl.ds(16, 16)] = rv[r, pl.ds(16, 16)] + pv1
                    return acc

                lax.fori_loop(0, L // 2, add_body, 0)
                store_descr(c, b).start()
            return carry

        lax.fori_loop(0, n_chunks // NBUF, outer, 0)

        for b in range(NBUF):
            store_descr(n_chunks - NBUF + b, b).wait()

    return k


def kernel(x, token_table, pos_table):
    B, L = x.shape
    V, E = token_table.shape
    BF = B * L
    per_w = BF // NW
    n_chunks = per_w // C

    xf = x.reshape(BF).astype(jnp.int32)
    x2 = xf.reshape(BF // G, G)

    k = _tok_pos_kernel(BF, L, E, per_w, n_chunks)
    out = k(x2, token_table, pos_table)
    return out.reshape(B, L, E)
